# baseline (speedup 1.0000x reference)
import functools

import jax
import jax.numpy as jnp
from jax import lax
from jax.experimental import pallas as pl
from jax.experimental.pallas import tpu as pltpu
from jax.experimental.pallas import tpu_sc as plsc

PH, PW = 7, 7
QK = PH * PW
NW = 32
B_T = 32
NPAD = NW * B_T
CH = 112
NCH = (B_T * QK) // CH
MAGIC = 12582912.0


def _make_roi_pool(N, H, W, C):
    mesh = plsc.VectorSubcoreMesh(core_axis_name="c", subcore_axis_name="s")
    num_cores = mesh.num_cores

    @functools.partial(
        pl.kernel,
        out_type=jax.ShapeDtypeStruct((NPAD * QK, C), jnp.float32),
        mesh=mesh,
        scratch_types=[
            pltpu.VMEM((B_T,), jnp.float32),
            pltpu.VMEM((B_T,), jnp.float32),
            pltpu.VMEM((B_T,), jnp.float32),
            pltpu.VMEM((B_T,), jnp.float32),
            pltpu.VMEM((NCH, CH), jnp.int32),
            pltpu.VMEM((NCH, CH), jnp.int32),
            pltpu.VMEM((CH, C), jnp.float32),
            pltpu.VMEM((CH, C), jnp.float32),
            pltpu.SemaphoreType.DMA,
            pltpu.SemaphoreType.DMA,
            pltpu.SemaphoreType.DMA,
            pltpu.SemaphoreType.DMA,
        ],
    )
    def roi_pool(fm_hbm, x1_hbm, y1_hbm, x2_hbm, y2_hbm, out_hbm,
                 x1v, y1v, x2v, y2v, gidx, sidx, buf0, buf1,
                 gsem0, gsem1, osem0, osem1):
        wid = lax.axis_index("s") * num_cores + lax.axis_index("c")
        roi_base = wid * B_T

        pltpu.sync_copy(x1_hbm.at[pl.ds(roi_base, B_T)], x1v)
        pltpu.sync_copy(y1_hbm.at[pl.ds(roi_base, B_T)], y1v)
        pltpu.sync_copy(x2_hbm.at[pl.ds(roi_base, B_T)], x2v)
        pltpu.sync_copy(y2_hbm.at[pl.ds(roi_base, B_T)], y2v)

        lanes = lax.iota(jnp.int32, 16)
        magic = jnp.float32(MAGIC)

        def rnd_clip(f, hi):
            r = (f + magic) - magic
            r = jnp.minimum(jnp.maximum(r, jnp.float32(0.0)), jnp.float32(hi))
            return r.astype(jnp.int32)

        for g in range(B_T // 16):
            sl = pl.ds(g * 16, 16)
            x1 = x1v[sl]
            y1 = y1v[sl]
            x2 = x2v[sl]
            y2 = y2v[sl]
            stepw = (x2 - x1) / jnp.float32(PW)
            steph = (y2 - y1) / jnp.float32(PH)
            n_vec = roi_base + g * 16 + lanes
            n_cl = jnp.minimum(n_vec, N - 1)
            out0 = n_vec * QK
            wcol = [rnd_clip(x1 + jnp.float32(j) * stepw, W - 1)
                    for j in range(PW)]
            for i in range(PH):
                hrow = rnd_clip(y1 + jnp.float32(i) * steph, H - 1)
                hpart = (n_cl * H + hrow) * W
                for j in range(PW):
                    q = g * QK + i * PW + j
                    dst = pl.ds((q % PH) * 16, 16)
                    gidx[q // PH, dst] = hpart + wcol[j]
                    sidx[q // PH, dst] = out0 + (i * PW + j)

        bufs = (buf0, buf1)
        gsems = (gsem0, gsem1)
        osems = (osem0, osem1)
        store_d = {}
        gather_d = {
            0: pltpu.async_copy(fm_hbm.at[gidx.at[0]], bufs[0], gsems[0])
        }
        for c in range(NCH):
            b = c % 2
            if c + 1 < NCH:
                nb = (c + 1) % 2
                if c >= 1:
                    store_d[c - 1].wait()
                gather_d[c + 1] = pltpu.async_copy(
                    fm_hbm.at[gidx.at[c + 1]], bufs[nb], gsems[nb])
            gather_d[c].wait()
            store_d[c] = pltpu.async_copy(
                bufs[b], out_hbm.at[sidx.at[c]], osems[b])
        store_d[NCH - 2].wait()
        store_d[NCH - 1].wait()

    return roi_pool


def kernel(feature_map, rois):
    N, H, W, C = feature_map.shape
    fm2d = feature_map.reshape(N * H * W, C)
    r = rois.astype(jnp.float32)
    pad = NPAD - N
    x1 = jnp.pad(r[:, 0], (0, pad))
    y1 = jnp.pad(r[:, 1], (0, pad))
    x2 = jnp.pad(r[:, 2], (0, pad))
    y2 = jnp.pad(r[:, 3], (0, pad))
    out = _make_roi_pool(N, H, W, C)(fm2d, x1, y1, x2, y2)
    return out.reshape(NPAD, PH, PW, C)[:N]


# speedup vs baseline: 1.0005x; 1.0005x over previous
import functools

import jax
import jax.numpy as jnp
from jax import lax
from jax.experimental import pallas as pl
from jax.experimental.pallas import tpu as pltpu
from jax.experimental.pallas import tpu_sc as plsc

PH, PW = 7, 7
QK = PH * PW
NW = 32
B_T = 32
NPAD = NW * B_T
CH = 112
NCH = (B_T * QK) // CH
NBUF = 4
SLAG = 2
MAGIC = 12582912.0


def _make_roi_pool(N, H, W, C):
    mesh = plsc.VectorSubcoreMesh(core_axis_name="c", subcore_axis_name="s")
    num_cores = mesh.num_cores

    @functools.partial(
        pl.kernel,
        out_type=jax.ShapeDtypeStruct((NPAD * QK, C), jnp.float32),
        mesh=mesh,
        scratch_types=[
            pltpu.VMEM((B_T,), jnp.float32),
            pltpu.VMEM((B_T,), jnp.float32),
            pltpu.VMEM((B_T,), jnp.float32),
            pltpu.VMEM((B_T,), jnp.float32),
            pltpu.VMEM((NCH, CH), jnp.int32),
            pltpu.VMEM((NCH, CH), jnp.int32),
            [pltpu.VMEM((CH, C), jnp.float32) for _ in range(NBUF)],
            [pltpu.SemaphoreType.DMA for _ in range(NBUF)],
            [pltpu.SemaphoreType.DMA for _ in range(NBUF)],
            pltpu.SemaphoreType.DMA,
        ],
    )
    def roi_pool(fm_hbm, x1_hbm, y1_hbm, x2_hbm, y2_hbm, out_hbm,
                 x1v, y1v, x2v, y2v, gidx, sidx, bufs, gsems, osems, rsem):
        wid = lax.axis_index("s") * num_cores + lax.axis_index("c")
        roi_base = wid * B_T

        cps = [
            pltpu.async_copy(x1_hbm.at[pl.ds(roi_base, B_T)], x1v, rsem),
            pltpu.async_copy(y1_hbm.at[pl.ds(roi_base, B_T)], y1v, rsem),
            pltpu.async_copy(x2_hbm.at[pl.ds(roi_base, B_T)], x2v, rsem),
            pltpu.async_copy(y2_hbm.at[pl.ds(roi_base, B_T)], y2v, rsem),
        ]
        for cp in cps:
            cp.wait()

        lanes = lax.iota(jnp.int32, 16)
        magic = jnp.float32(MAGIC)

        def rnd_clip(f, hi):
            r = (f + magic) - magic
            r = jnp.minimum(jnp.maximum(r, jnp.float32(0.0)), jnp.float32(hi))
            return r.astype(jnp.int32)

        gather_d = {}
        store_d = {}

        def start_store(c):
            gather_d[c].wait()
            store_d[c] = pltpu.async_copy(
                bufs[c % NBUF], out_hbm.at[sidx.at[c]], osems[c % NBUF])

        for g in range(B_T // 16):
            sl = pl.ds(g * 16, 16)
            x1 = x1v[sl]
            y1 = y1v[sl]
            x2 = x2v[sl]
            y2 = y2v[sl]
            stepw = (x2 - x1) / jnp.float32(PW)
            steph = (y2 - y1) / jnp.float32(PH)
            n_vec = roi_base + g * 16 + lanes
            n_cl = jnp.minimum(n_vec, N - 1)
            out0 = n_vec * QK
            wcol = [rnd_clip(x1 + jnp.float32(j) * stepw, W - 1)
                    for j in range(PW)]
            for i in range(PH):
                c = g * PH + i
                hrow = rnd_clip(y1 + jnp.float32(i) * steph, H - 1)
                hpart = (n_cl * H + hrow) * W
                for j in range(PW):
                    dst = pl.ds(j * 16, 16)
                    gidx[c, dst] = hpart + wcol[j]
                    sidx[c, dst] = out0 + (i * PW + j)
                if c >= NBUF:
                    store_d[c - NBUF].wait()
                gather_d[c] = pltpu.async_copy(
                    fm_hbm.at[gidx.at[c]], bufs[c % NBUF], gsems[c % NBUF])
                if c >= SLAG:
                    start_store(c - SLAG)
        for c in range(NCH - SLAG, NCH):
            start_store(c)
        for c in range(NCH - NBUF, NCH):
            store_d[c].wait()

    return roi_pool


def kernel(feature_map, rois):
    N, H, W, C = feature_map.shape
    fm2d = feature_map.reshape(N * H * W, C)
    r = rois.astype(jnp.float32)
    pad = NPAD - N
    x1 = jnp.pad(r[:, 0], (0, pad))
    y1 = jnp.pad(r[:, 1], (0, pad))
    x2 = jnp.pad(r[:, 2], (0, pad))
    y2 = jnp.pad(r[:, 3], (0, pad))
    out = _make_roi_pool(N, H, W, C)(fm2d, x1, y1, x2, y2)
    return out.reshape(NPAD, PH, PW, C)[:N]
